# baseline (speedup 1.0000x reference)
import functools

import jax
import jax.numpy as jnp
from jax import lax
from jax.experimental import pallas as pl
from jax.experimental.pallas import tpu as pltpu
from jax.experimental.pallas import tpu_sc as plsc

BATCH = 16384
D_FEAT = 128
EMBED_SIZE = 64

NC = 2
NS = 16
NW = NC * NS
B_PER_W = BATCH // NW
IDX_CHUNK = 128
N_CHUNK = B_PER_W // IDX_CHUNK

_SC_MESH = plsc.VectorSubcoreMesh(core_axis_name="c", subcore_axis_name="s",
                                  num_cores=NC, num_subcores=NS)


def _make_row_gather(width):
    def body(ids_hbm, tab_hbm, out_hbm, idx_v, rows_v, sem):
        wid = lax.axis_index("s") * NC + lax.axis_index("c")
        base = wid * B_PER_W
        pltpu.sync_copy(ids_hbm.at[wid], idx_v)
        for j in range(N_CHUNK):
            pltpu.async_copy(tab_hbm.at[idx_v.at[j]],
                             rows_v.at[pl.ds(j * IDX_CHUNK, IDX_CHUNK)], sem)
        for j in range(N_CHUNK):
            pltpu.make_async_copy(
                tab_hbm.at[idx_v.at[j]],
                rows_v.at[pl.ds(j * IDX_CHUNK, IDX_CHUNK)], sem).wait()
        pltpu.sync_copy(rows_v, out_hbm.at[pl.ds(base, B_PER_W)])

    return pl.kernel(
        body,
        out_type=jax.ShapeDtypeStruct((BATCH, width), jnp.float32),
        mesh=_SC_MESH,
        compiler_params=pltpu.CompilerParams(use_tc_tiling_on_sc=False),
        scratch_types=[
            pltpu.VMEM((N_CHUNK, IDX_CHUNK), jnp.int32),
            pltpu.VMEM((B_PER_W, width), jnp.float32),
            pltpu.SemaphoreType.DMA,
        ],
    )


_gather_feat = _make_row_gather(D_FEAT)
_gather_pairs = _make_row_gather(2 * EMBED_SIZE)


def _proj_body(x_ref, w_ref, b_ref, o_ref):
    o_ref[...] = (jnp.dot(w_ref[...], x_ref[...].T,
                          preferred_element_type=jnp.float32) + b_ref[...])


_ROWS_PER_BLK = 2048


def _tc_proj(x, w, b2d):
    return pl.pallas_call(
        _proj_body,
        grid=(BATCH // _ROWS_PER_BLK,),
        in_specs=[
            pl.BlockSpec((_ROWS_PER_BLK, D_FEAT), lambda i: (i, 0)),
            pl.BlockSpec((EMBED_SIZE, D_FEAT), lambda i: (0, 0)),
            pl.BlockSpec((EMBED_SIZE, 1), lambda i: (0, 0)),
        ],
        out_specs=pl.BlockSpec((EMBED_SIZE, _ROWS_PER_BLK), lambda i: (0, i)),
        out_shape=jax.ShapeDtypeStruct((EMBED_SIZE, BATCH), jnp.float32),
    )(x, w, b2d)


def _half_body(x_ref, p_ref, o_ref):
    lo = x_ref[:, :EMBED_SIZE]
    hi = x_ref[:, EMBED_SIZE:]
    o_ref[...] = lo + p_ref[...] * (hi - lo)


def _tc_half(pairs, parity):
    return pl.pallas_call(
        _half_body,
        grid=(BATCH // _ROWS_PER_BLK,),
        in_specs=[
            pl.BlockSpec((_ROWS_PER_BLK, 2 * EMBED_SIZE), lambda i: (i, 0)),
            pl.BlockSpec((_ROWS_PER_BLK, 1), lambda i: (i, 0)),
        ],
        out_specs=pl.BlockSpec((_ROWS_PER_BLK, EMBED_SIZE), lambda i: (i, 0)),
        out_shape=jax.ShapeDtypeStruct((BATCH, EMBED_SIZE), jnp.float32),
    )(pairs, parity)


def kernel(node_ids_feat, node_ids_embed, feat_table, proj_W, proj_b,
           embed_table):
    ids_f = node_ids_feat.astype(jnp.int32).reshape(NW, N_CHUNK, IDX_CHUNK)
    ids_e = node_ids_embed.astype(jnp.int32)
    ids_pair = (ids_e >> 1).reshape(NW, N_CHUNK, IDX_CHUNK)
    parity = (ids_e & 1).astype(jnp.float32).reshape(BATCH, 1)
    lin128 = embed_table.reshape(embed_table.shape[0] // 2,
                                 2 * EMBED_SIZE) * jnp.float32(1.0)
    pairs = _gather_pairs(ids_pair, lin128)
    gathered = _gather_feat(ids_f, feat_table)
    emb_embed = _tc_half(pairs, parity)
    feat_T = _tc_proj(gathered, proj_W, proj_b.reshape(EMBED_SIZE, 1))
    return (feat_T.T, emb_embed)


# speedup vs baseline: 1.5982x; 1.5982x over previous
import functools

import jax
import jax.numpy as jnp
from jax import lax
from jax.experimental import pallas as pl
from jax.experimental.pallas import tpu as pltpu
from jax.experimental.pallas import tpu_sc as plsc

BATCH = 16384
D_FEAT = 128
EMBED_SIZE = 64
N_EMB = 1000000

NC = 2
NS = 16
NW = NC * NS
B_PER_W = BATCH // NW
IDX_CHUNK = 128
N_CHUNK = B_PER_W // IDX_CHUNK

LANE = 128
N_BLOCKS = (N_EMB + LANE - 1) // LANE
BLOCKS_PER_TILE = (N_BLOCKS + NW - 1) // NW
TAIL_BLOCK = N_EMB // LANE
N_VECS = BATCH // 16
OROWS = 128
DUMP_BASE = BATCH
OUT_ROWS = BATCH + OROWS

_SC_MESH = plsc.VectorSubcoreMesh(core_axis_name="c", subcore_axis_name="s",
                                  num_cores=NC, num_subcores=NS)


def _make_row_gather(width):
    def body(ids_hbm, tab_hbm, out_hbm, idx_v, rows_v, sem):
        wid = lax.axis_index("s") * NC + lax.axis_index("c")
        base = wid * B_PER_W
        pltpu.sync_copy(ids_hbm.at[wid], idx_v)
        for j in range(N_CHUNK):
            pltpu.async_copy(tab_hbm.at[idx_v.at[j]],
                             rows_v.at[pl.ds(j * IDX_CHUNK, IDX_CHUNK)], sem)
        for j in range(N_CHUNK):
            pltpu.make_async_copy(
                tab_hbm.at[idx_v.at[j]],
                rows_v.at[pl.ds(j * IDX_CHUNK, IDX_CHUNK)], sem).wait()
        pltpu.sync_copy(rows_v, out_hbm.at[pl.ds(base, B_PER_W)])

    return pl.kernel(
        body,
        out_type=jax.ShapeDtypeStruct((BATCH, width), jnp.float32),
        mesh=_SC_MESH,
        compiler_params=pltpu.CompilerParams(use_tc_tiling_on_sc=False),
        scratch_types=[
            pltpu.VMEM((N_CHUNK, IDX_CHUNK), jnp.int32),
            pltpu.VMEM((B_PER_W, width), jnp.float32),
            pltpu.SemaphoreType.DMA,
        ],
    )


_gather_feat = _make_row_gather(D_FEAT)


def _emb_body(ids_hbm, embT_hbm, tail_hbm, out_hbm,
              ids_v, sel_id, sel_pos, cbuf, obuf, oidx, sem_c, sem_o):
    wid = lax.axis_index("s") * NC + lax.axis_index("c")
    blk_lo = wid * BLOCKS_PER_TILE
    blk_hi = jnp.minimum(blk_lo + BLOCKS_PER_TILE, N_BLOCKS)
    iota = lax.iota(jnp.int32, 16)

    pltpu.sync_copy(ids_hbm, ids_v)

    def select(v, ptr):
        ids16 = plsc.load_gather(ids_v, [v * 16 + iota])
        blk = lax.shift_right_logical(ids16, 7)
        m = (blk >= blk_lo) & (blk < blk_hi)
        mi = m.astype(jnp.int32)
        rank = plsc.cumsum(mi)
        dst = rank + (ptr - 1)
        plsc.store_scatter(sel_id, [dst], ids16, mask=m)
        plsc.store_scatter(sel_pos, [dst], v * 16 + iota, mask=m)
        return ptr + jnp.sum(mi)

    my_count = lax.fori_loop(0, N_VECS, select, jnp.int32(0))
    n_vec = (my_count + 15) // 16

    def init_oidx():
        for v in range(OROWS // 16):
            plsc.store_scatter(oidx, [v * 16 + iota],
                               DUMP_BASE + v * 16 + iota)

    init_oidx()

    def fire(blk, buf):
        safe = jnp.minimum(blk, TAIL_BLOCK - 1)

        @pl.when(blk < TAIL_BLOCK)
        def _():
            off = pl.multiple_of(safe * LANE, LANE)
            pltpu.async_copy(embT_hbm.at[:, pl.ds(off, LANE)],
                             cbuf.at[buf], sem_c)

        @pl.when(blk >= TAIL_BLOCK)
        def _():
            pltpu.async_copy(tail_hbm, cbuf.at[buf], sem_c)

    def wait_block(blk, buf):
        safe = jnp.minimum(blk, TAIL_BLOCK - 1)
        off = pl.multiple_of(safe * LANE, LANE)
        pltpu.make_async_copy(embT_hbm.at[:, pl.ds(off, LANE)],
                              cbuf.at[buf], sem_c).wait()

    def flush(row):
        cp = pltpu.async_copy(obuf, out_hbm.at[oidx], sem_o)
        cp.wait()
        init_oidx()

    fire(blk_lo, 0)

    def per_block(b, row):
        blk = blk_lo + b
        buf = lax.rem(b, 2)

        @pl.when(blk + 1 < blk_hi)
        def _():
            fire(blk + 1, lax.rem(b + 1, 2))

        wait_block(blk, buf)

        def per_vec(v, row):
            @pl.when(row + 16 > OROWS)
            def _():
                flush(row)

            row = jnp.where(row + 16 > OROWS, 0, row)
            ids16 = plsc.load_gather(sel_id, [v * 16 + iota])
            valid = (v * 16 + iota) < my_count
            m = ((lax.shift_right_logical(ids16, 7) == blk) & valid)
            cnt = jnp.sum(m.astype(jnp.int32))

            def per_hit(_, carry):
                m, row = carry
                j = plsc.all_reduce_ffs(m)
                idv = plsc.load_gather(sel_id, [v * 16 + j])
                psv = plsc.load_gather(sel_pos, [v * 16 + j])
                lane = idv & (LANE - 1)
                rowv = jnp.full((16,), row, jnp.int32)
                for s in range(EMBED_SIZE // 16):
                    vals = plsc.load_gather(
                        cbuf, [jnp.full((16,), buf, jnp.int32),
                               s * 16 + iota, lane])
                    plsc.store_scatter(obuf,
                                       [rowv, s * 16 + iota], vals)
                lane0 = iota == 0
                plsc.store_scatter(oidx, [jnp.full((16,), row, jnp.int32)],
                                   psv, mask=lane0)
                m = m & (~(iota == j))
                return m, row + 1

            _, row = lax.fori_loop(0, cnt, per_hit, (m, row))
            return row

        return lax.fori_loop(0, n_vec, per_vec, row)

    row = lax.fori_loop(0, blk_hi - blk_lo, per_block, jnp.int32(0))
    flush(row)


_emb_gather = pl.kernel(
    _emb_body,
    out_type=jax.ShapeDtypeStruct((OUT_ROWS, LANE), jnp.float32),
    mesh=_SC_MESH,
    compiler_params=pltpu.CompilerParams(needs_layout_passes=False),
    scratch_types=[
        pltpu.VMEM((BATCH,), jnp.int32),
        pltpu.VMEM((BATCH,), jnp.int32),
        pltpu.VMEM((BATCH,), jnp.int32),
        pltpu.VMEM((2, EMBED_SIZE, LANE), jnp.float32),
        pltpu.VMEM((OROWS, LANE), jnp.float32),
        pltpu.VMEM((OROWS,), jnp.int32),
        pltpu.SemaphoreType.DMA,
        pltpu.SemaphoreType.DMA,
    ],
)


def _proj_body(x_ref, w_ref, b_ref, o_ref):
    o_ref[...] = (jnp.dot(w_ref[...], x_ref[...].T,
                          preferred_element_type=jnp.float32) + b_ref[...])


_ROWS_PER_BLK = 2048


def _tc_proj(x, w, b2d):
    return pl.pallas_call(
        _proj_body,
        grid=(BATCH // _ROWS_PER_BLK,),
        in_specs=[
            pl.BlockSpec((_ROWS_PER_BLK, D_FEAT), lambda i: (i, 0)),
            pl.BlockSpec((EMBED_SIZE, D_FEAT), lambda i: (0, 0)),
            pl.BlockSpec((EMBED_SIZE, 1), lambda i: (0, 0)),
        ],
        out_specs=pl.BlockSpec((EMBED_SIZE, _ROWS_PER_BLK), lambda i: (0, i)),
        out_shape=jax.ShapeDtypeStruct((EMBED_SIZE, BATCH), jnp.float32),
    )(x, w, b2d)


def kernel(node_ids_feat, node_ids_embed, feat_table, proj_W, proj_b,
           embed_table):
    ids_f = node_ids_feat.astype(jnp.int32).reshape(NW, N_CHUNK, IDX_CHUNK)
    ids_e = node_ids_embed.astype(jnp.int32)
    emb_T = embed_table.T
    tail = jnp.zeros((EMBED_SIZE, LANE), jnp.float32)
    tail = tail.at[:, :N_EMB - TAIL_BLOCK * LANE].set(
        emb_T[:, TAIL_BLOCK * LANE:])
    out_pad = _emb_gather(ids_e, emb_T, tail)
    gathered = _gather_feat(ids_f, feat_table)
    feat_T = _tc_proj(gathered, proj_W, proj_b.reshape(EMBED_SIZE, 1))
    return (feat_T.T, out_pad[:BATCH, :EMBED_SIZE])


# speedup vs baseline: 2.8473x; 1.7816x over previous
import functools

import jax
import jax.numpy as jnp
from jax import lax
from jax.experimental import pallas as pl
from jax.experimental.pallas import tpu as pltpu
from jax.experimental.pallas import tpu_sc as plsc

BATCH = 16384
D_FEAT = 128
EMBED_SIZE = 64
N_EMB = 1000000

NC = 2
NS = 16
NW = NC * NS
B_PER_W = BATCH // NW
IDX_CHUNK = 128
N_CHUNK = B_PER_W // IDX_CHUNK

LANE = 128
N_BLOCKS = (N_EMB + LANE - 1) // LANE
BLOCKS_PER_TILE = (N_BLOCKS + NW - 1) // NW
TAIL_BLOCK = N_EMB // LANE
N_VECS = BATCH // 16
GRP_SHIFT = 4
NGRP = 16
OROWS = 128
DUMP_BASE = BATCH
OUT_ROWS = BATCH + OROWS

_SC_MESH = plsc.VectorSubcoreMesh(core_axis_name="c", subcore_axis_name="s",
                                  num_cores=NC, num_subcores=NS)


def _make_row_gather(width):
    def body(ids_hbm, tab_hbm, out_hbm, idx_v, rows_v, sem):
        wid = lax.axis_index("s") * NC + lax.axis_index("c")
        base = wid * B_PER_W
        pltpu.sync_copy(ids_hbm.at[wid], idx_v)
        for j in range(N_CHUNK):
            pltpu.async_copy(tab_hbm.at[idx_v.at[j]],
                             rows_v.at[pl.ds(j * IDX_CHUNK, IDX_CHUNK)], sem)
        for j in range(N_CHUNK):
            pltpu.make_async_copy(
                tab_hbm.at[idx_v.at[j]],
                rows_v.at[pl.ds(j * IDX_CHUNK, IDX_CHUNK)], sem).wait()
        pltpu.sync_copy(rows_v, out_hbm.at[pl.ds(base, B_PER_W)])

    return pl.kernel(
        body,
        out_type=jax.ShapeDtypeStruct((BATCH, width), jnp.float32),
        mesh=_SC_MESH,
        compiler_params=pltpu.CompilerParams(use_tc_tiling_on_sc=False),
        scratch_types=[
            pltpu.VMEM((N_CHUNK, IDX_CHUNK), jnp.int32),
            pltpu.VMEM((B_PER_W, width), jnp.float32),
            pltpu.SemaphoreType.DMA,
        ],
    )


_gather_feat = _make_row_gather(D_FEAT)


def _emb_body(ids_hbm, embT_hbm, tail_hbm, out_hbm,
              ids_v, sel_id, sel_pos, sel2_pos, goff_s,
              cbuf, obuf, oidx, sem_c, sem_o):
    wid = lax.axis_index("s") * NC + lax.axis_index("c")
    blk_lo = wid * BLOCKS_PER_TILE
    blk_hi = jnp.minimum(blk_lo + BLOCKS_PER_TILE, N_BLOCKS)
    iota = lax.iota(jnp.int32, 16)

    pltpu.sync_copy(ids_hbm, ids_v)

    def select(v, ptr):
        ids16 = plsc.load_gather(ids_v, [v * 16 + iota])
        blk = lax.shift_right_logical(ids16, 7)
        m = (blk >= blk_lo) & (blk < blk_hi)
        mi = m.astype(jnp.int32)
        rank = plsc.cumsum(mi)
        dst = rank + (ptr - 1)
        plsc.store_scatter(sel_id, [dst], ids16, mask=m)
        plsc.store_scatter(sel_pos, [dst], v * 16 + iota, mask=m)
        return ptr + jnp.sum(mi)

    my_count = lax.fori_loop(0, N_VECS, select, jnp.int32(0))
    n_vec = (my_count + 15) // 16

    def count_grp(v, cnts):
        ids16 = plsc.load_gather(sel_id, [v * 16 + iota])
        valid = (v * 16 + iota) < my_count
        g16 = lax.shift_right_logical(
            lax.shift_right_logical(ids16, 7) - blk_lo, GRP_SHIFT)
        out = []
        for g in range(NGRP):
            m = (g16 == g) & valid
            out.append(cnts[g] + jnp.sum(m.astype(jnp.int32)))
        return tuple(out)

    cnts = lax.fori_loop(0, n_vec, count_grp,
                         tuple(jnp.int32(0) for _ in range(NGRP)))
    offs = []
    acc = jnp.int32(0)
    for g in range(NGRP):
        offs.append(acc)
        goff_s[g] = acc
        acc = acc + cnts[g]
    goff_s[NGRP] = acc

    def place_grp(v, ptrs):
        ids16 = plsc.load_gather(sel_id, [v * 16 + iota])
        pos16 = plsc.load_gather(sel_pos, [v * 16 + iota])
        valid = (v * 16 + iota) < my_count
        g16 = lax.shift_right_logical(
            lax.shift_right_logical(ids16, 7) - blk_lo, GRP_SHIFT)
        new = []
        for g in range(NGRP):
            m = (g16 == g) & valid
            mi = m.astype(jnp.int32)
            rank = plsc.cumsum(mi)
            dst = rank + (ptrs[g] - 1)
            plsc.store_scatter(ids_v, [dst], ids16, mask=m)
            plsc.store_scatter(sel2_pos, [dst], pos16, mask=m)
            new.append(ptrs[g] + jnp.sum(mi))
        return tuple(new)

    lax.fori_loop(0, n_vec, place_grp, tuple(offs))

    def init_oidx():
        for v in range(OROWS // 16):
            plsc.store_scatter(oidx, [v * 16 + iota],
                               DUMP_BASE + v * 16 + iota)

    init_oidx()

    def fire(blk, buf):
        safe = jnp.minimum(blk, TAIL_BLOCK - 1)

        @pl.when(blk < TAIL_BLOCK)
        def _():
            off = pl.multiple_of(safe * LANE, LANE)
            pltpu.async_copy(embT_hbm.at[:, pl.ds(off, LANE)],
                             cbuf.at[buf], sem_c)

        @pl.when(blk >= TAIL_BLOCK)
        def _():
            pltpu.async_copy(tail_hbm, cbuf.at[buf], sem_c)

    def wait_block(blk, buf):
        safe = jnp.minimum(blk, TAIL_BLOCK - 1)
        off = pl.multiple_of(safe * LANE, LANE)
        pltpu.make_async_copy(embT_hbm.at[:, pl.ds(off, LANE)],
                              cbuf.at[buf], sem_c).wait()

    def flush(row):
        cp = pltpu.async_copy(obuf, out_hbm.at[oidx], sem_o)
        cp.wait()
        init_oidx()

    fire(blk_lo, 0)

    def per_block(b, row):
        blk = blk_lo + b
        buf = lax.rem(b, 2)

        @pl.when(blk + 1 < blk_hi)
        def _():
            fire(blk + 1, lax.rem(b + 1, 2))

        wait_block(blk, buf)

        g = lax.shift_right_logical(b, GRP_SHIFT)
        lo_g = goff_s[g]
        hi_g = goff_s[g + 1]

        def per_vec(v, row):
            @pl.when(row + 16 > OROWS)
            def _():
                flush(row)

            row = jnp.where(row + 16 > OROWS, 0, row)
            p16 = v * 16 + iota
            ids16 = plsc.load_gather(ids_v, [p16])
            valid = (p16 >= lo_g) & (p16 < hi_g)
            m = ((lax.shift_right_logical(ids16, 7) == blk) & valid)
            cnt = jnp.sum(m.astype(jnp.int32))

            def per_hit(_, carry):
                m, row = carry
                j = plsc.all_reduce_ffs(m)
                idv = plsc.load_gather(ids_v, [v * 16 + j])
                psv = plsc.load_gather(sel2_pos, [v * 16 + j])
                lane = idv & (LANE - 1)
                rowv = jnp.full((16,), row, jnp.int32)
                for s in range(EMBED_SIZE // 16):
                    vals = plsc.load_gather(
                        cbuf, [jnp.full((16,), buf, jnp.int32),
                               s * 16 + iota, lane])
                    plsc.store_scatter(obuf,
                                       [rowv, s * 16 + iota], vals)
                lane0 = iota == 0
                plsc.store_scatter(oidx, [jnp.full((16,), row, jnp.int32)],
                                   psv, mask=lane0)
                m = m & (~(iota == j))
                return m, row + 1

            _, row = lax.fori_loop(0, cnt, per_hit, (m, row))
            return row

        return lax.fori_loop(lo_g >> 4, (hi_g + 15) >> 4, per_vec, row)

    row = lax.fori_loop(0, blk_hi - blk_lo, per_block, jnp.int32(0))
    flush(row)


_emb_gather = pl.kernel(
    _emb_body,
    out_type=jax.ShapeDtypeStruct((OUT_ROWS, LANE), jnp.float32),
    mesh=_SC_MESH,
    compiler_params=pltpu.CompilerParams(needs_layout_passes=False),
    scratch_types=[
        pltpu.VMEM((BATCH,), jnp.int32),
        pltpu.VMEM((BATCH,), jnp.int32),
        pltpu.VMEM((BATCH,), jnp.int32),
        pltpu.VMEM((BATCH,), jnp.int32),
        pltpu.SMEM((NGRP + 1,), jnp.int32),
        pltpu.VMEM((2, EMBED_SIZE, LANE), jnp.float32),
        pltpu.VMEM((OROWS, LANE), jnp.float32),
        pltpu.VMEM((OROWS,), jnp.int32),
        pltpu.SemaphoreType.DMA,
        pltpu.SemaphoreType.DMA,
    ],
)


def _proj_body(x_ref, w_ref, b_ref, o_ref):
    o_ref[...] = (jnp.dot(w_ref[...], x_ref[...].T,
                          preferred_element_type=jnp.float32) + b_ref[...])


_ROWS_PER_BLK = 2048


def _tc_proj(x, w, b2d):
    return pl.pallas_call(
        _proj_body,
        grid=(BATCH // _ROWS_PER_BLK,),
        in_specs=[
            pl.BlockSpec((_ROWS_PER_BLK, D_FEAT), lambda i: (i, 0)),
            pl.BlockSpec((EMBED_SIZE, D_FEAT), lambda i: (0, 0)),
            pl.BlockSpec((EMBED_SIZE, 1), lambda i: (0, 0)),
        ],
        out_specs=pl.BlockSpec((EMBED_SIZE, _ROWS_PER_BLK), lambda i: (0, i)),
        out_shape=jax.ShapeDtypeStruct((EMBED_SIZE, BATCH), jnp.float32),
    )(x, w, b2d)


def kernel(node_ids_feat, node_ids_embed, feat_table, proj_W, proj_b,
           embed_table):
    ids_f = node_ids_feat.astype(jnp.int32).reshape(NW, N_CHUNK, IDX_CHUNK)
    ids_e = node_ids_embed.astype(jnp.int32)
    emb_T = embed_table.T
    tail = jnp.zeros((EMBED_SIZE, LANE), jnp.float32)
    tail = tail.at[:, :N_EMB - TAIL_BLOCK * LANE].set(
        emb_T[:, TAIL_BLOCK * LANE:])
    out_pad = _emb_gather(ids_e, emb_T, tail)
    gathered = _gather_feat(ids_f, feat_table)
    feat_T = _tc_proj(gathered, proj_W, proj_b.reshape(EMBED_SIZE, 1))
    return (feat_T.T, out_pad[:BATCH, :EMBED_SIZE])


# speedup vs baseline: 3.8624x; 1.3565x over previous
import functools

import jax
import jax.numpy as jnp
from jax import lax
from jax.experimental import pallas as pl
from jax.experimental.pallas import tpu as pltpu
from jax.experimental.pallas import tpu_sc as plsc

BATCH = 16384
D_FEAT = 128
EMBED_SIZE = 64
N_EMB = 1000000

NC = 2
NS = 16
NW = NC * NS
B_PER_W = BATCH // NW
IDX_CHUNK = 128
N_CHUNK = B_PER_W // IDX_CHUNK

LANE = 128
N_BLOCKS = (N_EMB + LANE - 1) // LANE
BLOCKS_PER_TILE = (N_BLOCKS + NW - 1) // NW
TAIL_BLOCK = N_EMB // LANE
N_VECS = BATCH // 16
GRP_SHIFT = 4
NGRP = 16
OROWS = 128
DUMP_BASE = BATCH
OUT_ROWS = BATCH + OROWS

_SC_MESH = plsc.VectorSubcoreMesh(core_axis_name="c", subcore_axis_name="s",
                                  num_cores=NC, num_subcores=NS)


def _make_row_gather(width):
    def body(ids_hbm, tab_hbm, out_hbm, idx_v, rows_v, sem):
        wid = lax.axis_index("s") * NC + lax.axis_index("c")
        base = wid * B_PER_W
        pltpu.sync_copy(ids_hbm.at[wid], idx_v)
        for j in range(N_CHUNK):
            pltpu.async_copy(tab_hbm.at[idx_v.at[j]],
                             rows_v.at[pl.ds(j * IDX_CHUNK, IDX_CHUNK)], sem)
        for j in range(N_CHUNK):
            pltpu.make_async_copy(
                tab_hbm.at[idx_v.at[j]],
                rows_v.at[pl.ds(j * IDX_CHUNK, IDX_CHUNK)], sem).wait()
        pltpu.sync_copy(rows_v, out_hbm.at[pl.ds(base, B_PER_W)])

    return pl.kernel(
        body,
        out_type=jax.ShapeDtypeStruct((BATCH, width), jnp.float32),
        mesh=_SC_MESH,
        compiler_params=pltpu.CompilerParams(use_tc_tiling_on_sc=False),
        scratch_types=[
            pltpu.VMEM((N_CHUNK, IDX_CHUNK), jnp.int32),
            pltpu.VMEM((B_PER_W, width), jnp.float32),
            pltpu.SemaphoreType.DMA,
        ],
    )


_gather_feat = _make_row_gather(D_FEAT)


def _emb_body(ids_hbm, embT_hbm, tail_hbm, out_hbm,
              ids_v, sel_id, sel_pos, sel2_pos, goff_s, boff_s, blist_s,
              cbuf, obuf, oidx, sem_c, sem_o):
    wid = lax.axis_index("s") * NC + lax.axis_index("c")
    blk_lo = wid * BLOCKS_PER_TILE
    blk_hi = jnp.minimum(blk_lo + BLOCKS_PER_TILE, N_BLOCKS)
    nblk = blk_hi - blk_lo
    iota = lax.iota(jnp.int32, 16)

    pltpu.sync_copy(ids_hbm, ids_v)

    def select(v, ptr):
        ids16 = plsc.load_gather(ids_v, [v * 16 + iota])
        blk = lax.shift_right_logical(ids16, 7)
        m = (blk >= blk_lo) & (blk < blk_hi)
        mi = m.astype(jnp.int32)
        rank = plsc.cumsum(mi)
        dst = rank + (ptr - 1)
        plsc.store_scatter(sel_id, [dst], ids16, mask=m)
        plsc.store_scatter(sel_pos, [dst], v * 16 + iota, mask=m)
        return ptr + jnp.sum(mi)

    my_count = lax.fori_loop(0, N_VECS, select, jnp.int32(0))
    n_vec = (my_count + 15) // 16

    def count_grp(v, cnts):
        ids16 = plsc.load_gather(sel_id, [v * 16 + iota])
        valid = (v * 16 + iota) < my_count
        g16 = lax.shift_right_logical(
            lax.shift_right_logical(ids16, 7) - blk_lo, GRP_SHIFT)
        return tuple(cnts[g] + jnp.sum(((g16 == g) & valid).astype(jnp.int32))
                     for g in range(NGRP))

    cnts = lax.fori_loop(0, n_vec, count_grp,
                         tuple(jnp.int32(0) for _ in range(NGRP)))
    offs = []
    acc = jnp.int32(0)
    for g in range(NGRP):
        offs.append(acc)
        goff_s[g] = acc
        acc = acc + cnts[g]
    goff_s[NGRP] = acc

    def place_grp(v, ptrs):
        ids16 = plsc.load_gather(sel_id, [v * 16 + iota])
        pos16 = plsc.load_gather(sel_pos, [v * 16 + iota])
        valid = (v * 16 + iota) < my_count
        g16 = lax.shift_right_logical(
            lax.shift_right_logical(ids16, 7) - blk_lo, GRP_SHIFT)
        new = []
        for g in range(NGRP):
            m = (g16 == g) & valid
            mi = m.astype(jnp.int32)
            rank = plsc.cumsum(mi)
            dst = rank + (ptrs[g] - 1)
            plsc.store_scatter(ids_v, [dst], ids16, mask=m)
            plsc.store_scatter(sel2_pos, [dst], pos16, mask=m)
            new.append(ptrs[g] + jnp.sum(mi))
        return tuple(new)

    lax.fori_loop(0, n_vec, place_grp, tuple(offs))

    BPG = 1 << GRP_SHIFT
    for g in range(NGRP):
        lo_g = goff_s[g]
        hi_g = goff_s[g + 1]

        def cnt_blk(v, cs, g=g, lo_g=lo_g, hi_g=hi_g):
            p16 = v * 16 + iota
            ids16 = plsc.load_gather(ids_v, [p16])
            valid = (p16 >= lo_g) & (p16 < hi_g)
            lb16 = lax.shift_right_logical(ids16, 7) - blk_lo
            return tuple(
                cs[j] + jnp.sum(((lb16 == g * BPG + j) & valid)
                                .astype(jnp.int32))
                for j in range(BPG))

        bc = lax.fori_loop(lo_g >> 4, (hi_g + 15) >> 4, cnt_blk,
                           tuple(jnp.int32(0) for _ in range(BPG)))
        for j in range(BPG):
            boff_s[g * BPG + j] = bc[j]

    def prefix(i, acc2):
        c = boff_s[i]
        boff_s[i] = acc2
        return acc2 + c

    total = lax.fori_loop(0, NGRP * BPG, prefix, jnp.int32(0))
    boff_s[NGRP * BPG] = total

    for g in range(NGRP):
        lo_g = goff_s[g]
        hi_g = goff_s[g + 1]
        ptr0 = tuple(boff_s[g * BPG + j] for j in range(BPG))

        def place_blk(v, ps, g=g, lo_g=lo_g, hi_g=hi_g):
            p16 = v * 16 + iota
            ids16 = plsc.load_gather(ids_v, [p16])
            pos16 = plsc.load_gather(sel2_pos, [p16])
            valid = (p16 >= lo_g) & (p16 < hi_g)
            lb16 = lax.shift_right_logical(ids16, 7) - blk_lo
            new = []
            for j in range(BPG):
                m = (lb16 == g * BPG + j) & valid
                mi = m.astype(jnp.int32)
                rank = plsc.cumsum(mi)
                dst = rank + (ps[j] - 1)
                plsc.store_scatter(sel_id, [dst], ids16, mask=m)
                plsc.store_scatter(sel_pos, [dst], pos16, mask=m)
                new.append(ps[j] + jnp.sum(mi))
            return tuple(new)

        lax.fori_loop(lo_g >> 4, (hi_g + 15) >> 4, place_blk, ptr0)

    def build_list(i, m):
        c = boff_s[i + 1] - boff_s[i]

        @pl.when((c > 0) & (i < nblk))
        def _():
            blist_s[m] = i

        return m + jnp.where((c > 0) & (i < nblk), 1, 0)

    n_live = lax.fori_loop(0, NGRP * BPG, build_list, jnp.int32(0))

    def init_oidx():
        for v in range(OROWS // 16):
            plsc.store_scatter(oidx, [v * 16 + iota],
                               DUMP_BASE + v * 16 + iota)

    init_oidx()

    def fire(blk, buf):
        safe = jnp.minimum(blk, TAIL_BLOCK - 1)

        @pl.when(blk < TAIL_BLOCK)
        def _():
            off = pl.multiple_of(safe * LANE, LANE)
            pltpu.async_copy(embT_hbm.at[:, pl.ds(off, LANE)],
                             cbuf.at[buf], sem_c)

        @pl.when(blk >= TAIL_BLOCK)
        def _():
            pltpu.async_copy(tail_hbm, cbuf.at[buf], sem_c)

    def wait_block(blk, buf):
        safe = jnp.minimum(blk, TAIL_BLOCK - 1)
        off = pl.multiple_of(safe * LANE, LANE)
        pltpu.make_async_copy(embT_hbm.at[:, pl.ds(off, LANE)],
                              cbuf.at[buf], sem_c).wait()

    def flush(row):
        cp = pltpu.async_copy(obuf, out_hbm.at[oidx], sem_o)
        cp.wait()
        init_oidx()

    RING = 4

    def prefire(n, _):
        fire(blk_lo + blist_s[n], lax.rem(n, RING))
        return _

    lax.fori_loop(0, jnp.minimum(n_live, RING), prefire, jnp.int32(0))

    def per_block(n, row):
        lb = blist_s[n]
        blk = blk_lo + lb
        slot = lax.rem(n, RING)
        wait_block(blk, slot)
        k_lo = boff_s[lb]
        k_hi = boff_s[lb + 1]

        def per_hit(k, row):
            @pl.when(row >= OROWS)
            def _():
                flush(row)

            row = jnp.where(row >= OROWS, 0, row)
            kv = jnp.full((16,), k, jnp.int32)
            idv = plsc.load_gather(sel_id, [kv])
            psv = plsc.load_gather(sel_pos, [kv])
            lane = idv & (LANE - 1)
            rowv = jnp.full((16,), row, jnp.int32)
            for s in range(EMBED_SIZE // 16):
                vals = plsc.load_gather(
                    cbuf, [jnp.full((16,), slot, jnp.int32),
                           s * 16 + iota, lane])
                plsc.store_scatter(obuf, [rowv, s * 16 + iota], vals)
            plsc.store_scatter(oidx, [rowv], psv, mask=iota == 0)
            return row + 1

        row = lax.fori_loop(k_lo, k_hi, per_hit, row)

        @pl.when(n + RING < n_live)
        def _():
            fire(blk_lo + blist_s[n + RING], slot)

        return row

    row = lax.fori_loop(0, n_live, per_block, jnp.int32(0))
    flush(row)


_emb_gather = pl.kernel(
    _emb_body,
    out_type=jax.ShapeDtypeStruct((OUT_ROWS, LANE), jnp.float32),
    mesh=_SC_MESH,
    compiler_params=pltpu.CompilerParams(needs_layout_passes=False),
    scratch_types=[
        pltpu.VMEM((BATCH,), jnp.int32),
        pltpu.VMEM((BATCH,), jnp.int32),
        pltpu.VMEM((BATCH,), jnp.int32),
        pltpu.VMEM((BATCH,), jnp.int32),
        pltpu.SMEM((NGRP + 1,), jnp.int32),
        pltpu.SMEM((NGRP * (1 << GRP_SHIFT) + 1,), jnp.int32),
        pltpu.SMEM((NGRP * (1 << GRP_SHIFT),), jnp.int32),
        pltpu.VMEM((4, EMBED_SIZE, LANE), jnp.float32),
        pltpu.VMEM((OROWS, LANE), jnp.float32),
        pltpu.VMEM((OROWS,), jnp.int32),
        pltpu.SemaphoreType.DMA,
        pltpu.SemaphoreType.DMA,
    ],
)


def _proj_body(x_ref, w_ref, b_ref, o_ref):
    o_ref[...] = (jnp.dot(w_ref[...], x_ref[...].T,
                          preferred_element_type=jnp.float32) + b_ref[...])


_ROWS_PER_BLK = 2048


def _tc_proj(x, w, b2d):
    return pl.pallas_call(
        _proj_body,
        grid=(BATCH // _ROWS_PER_BLK,),
        in_specs=[
            pl.BlockSpec((_ROWS_PER_BLK, D_FEAT), lambda i: (i, 0)),
            pl.BlockSpec((EMBED_SIZE, D_FEAT), lambda i: (0, 0)),
            pl.BlockSpec((EMBED_SIZE, 1), lambda i: (0, 0)),
        ],
        out_specs=pl.BlockSpec((EMBED_SIZE, _ROWS_PER_BLK), lambda i: (0, i)),
        out_shape=jax.ShapeDtypeStruct((EMBED_SIZE, BATCH), jnp.float32),
    )(x, w, b2d)


def kernel(node_ids_feat, node_ids_embed, feat_table, proj_W, proj_b,
           embed_table):
    ids_f = node_ids_feat.astype(jnp.int32).reshape(NW, N_CHUNK, IDX_CHUNK)
    ids_e = node_ids_embed.astype(jnp.int32)
    emb_T = embed_table.T
    tail = jnp.zeros((EMBED_SIZE, LANE), jnp.float32)
    tail = tail.at[:, :N_EMB - TAIL_BLOCK * LANE].set(
        emb_T[:, TAIL_BLOCK * LANE:])
    out_pad = _emb_gather(ids_e, emb_T, tail)
    gathered = _gather_feat(ids_f, feat_table)
    feat_T = _tc_proj(gathered, proj_W, proj_b.reshape(EMBED_SIZE, 1))
    return (feat_T.T, out_pad[:BATCH, :EMBED_SIZE])


# speedup vs baseline: 3.8733x; 1.0028x over previous
import functools

import jax
import jax.numpy as jnp
from jax import lax
from jax.experimental import pallas as pl
from jax.experimental.pallas import tpu as pltpu
from jax.experimental.pallas import tpu_sc as plsc

BATCH = 16384
D_FEAT = 128
EMBED_SIZE = 64
N_EMB = 1000000

NC = 2
NS = 16
NW = NC * NS
B_PER_W = BATCH // NW
IDX_CHUNK = 128
N_CHUNK = B_PER_W // IDX_CHUNK

LANE = 128
N_BLOCKS = (N_EMB + LANE - 1) // LANE
BLOCKS_PER_TILE = (N_BLOCKS + NW - 1) // NW
TAIL_BLOCK = N_EMB // LANE
N_VECS = BATCH // 16
GRP_SHIFT = 4
NGRP = 16
OROWS = 128
DUMP_BASE = BATCH
OUT_ROWS = BATCH + OROWS

_SC_MESH = plsc.VectorSubcoreMesh(core_axis_name="c", subcore_axis_name="s",
                                  num_cores=NC, num_subcores=NS)


def _make_row_gather(width):
    def body(ids_hbm, tab_hbm, out_hbm, idx_v, rows_v, sem):
        wid = lax.axis_index("s") * NC + lax.axis_index("c")
        base = wid * B_PER_W
        pltpu.sync_copy(ids_hbm.at[wid], idx_v)
        for j in range(N_CHUNK):
            pltpu.async_copy(tab_hbm.at[idx_v.at[j]],
                             rows_v.at[pl.ds(j * IDX_CHUNK, IDX_CHUNK)], sem)
        for j in range(N_CHUNK):
            pltpu.make_async_copy(
                tab_hbm.at[idx_v.at[j]],
                rows_v.at[pl.ds(j * IDX_CHUNK, IDX_CHUNK)], sem).wait()
        pltpu.sync_copy(rows_v, out_hbm.at[pl.ds(base, B_PER_W)])

    return pl.kernel(
        body,
        out_type=jax.ShapeDtypeStruct((BATCH, width), jnp.float32),
        mesh=_SC_MESH,
        compiler_params=pltpu.CompilerParams(use_tc_tiling_on_sc=False),
        scratch_types=[
            pltpu.VMEM((N_CHUNK, IDX_CHUNK), jnp.int32),
            pltpu.VMEM((B_PER_W, width), jnp.float32),
            pltpu.SemaphoreType.DMA,
        ],
    )


_gather_feat = _make_row_gather(D_FEAT)


def _emb_body(ids_hbm, embT_hbm, tail_hbm, out_hbm,
              ids_v, sel_id, sel_pos, sel2_pos, goff_s, boff_s, blist_s,
              cbuf, obuf, oidx, sem_c, sem_o):
    wid = lax.axis_index("s") * NC + lax.axis_index("c")
    blk_lo = wid * BLOCKS_PER_TILE
    blk_hi = jnp.minimum(blk_lo + BLOCKS_PER_TILE, N_BLOCKS)
    nblk = blk_hi - blk_lo
    iota = lax.iota(jnp.int32, 16)

    pltpu.sync_copy(ids_hbm, ids_v)

    def select(v, ptr):
        ids16 = plsc.load_gather(ids_v, [v * 16 + iota])
        blk = lax.shift_right_logical(ids16, 7)
        m = (blk >= blk_lo) & (blk < blk_hi)
        mi = m.astype(jnp.int32)
        rank = plsc.cumsum(mi)
        dst = rank + (ptr - 1)
        plsc.store_scatter(sel_id, [dst], ids16, mask=m)
        plsc.store_scatter(sel_pos, [dst], v * 16 + iota, mask=m)
        return ptr + jnp.sum(mi)

    my_count = lax.fori_loop(0, N_VECS, select, jnp.int32(0))
    n_vec = (my_count + 15) // 16

    def count_grp(v, cnts):
        ids16 = plsc.load_gather(sel_id, [v * 16 + iota])
        valid = (v * 16 + iota) < my_count
        g16 = lax.shift_right_logical(
            lax.shift_right_logical(ids16, 7) - blk_lo, GRP_SHIFT)
        return tuple(cnts[g] + jnp.sum(((g16 == g) & valid).astype(jnp.int32))
                     for g in range(NGRP))

    cnts = lax.fori_loop(0, n_vec, count_grp,
                         tuple(jnp.int32(0) for _ in range(NGRP)))
    offs = []
    acc = jnp.int32(0)
    for g in range(NGRP):
        offs.append(acc)
        goff_s[g] = acc
        acc = acc + cnts[g]
    goff_s[NGRP] = acc

    def place_grp(v, ptrs):
        ids16 = plsc.load_gather(sel_id, [v * 16 + iota])
        pos16 = plsc.load_gather(sel_pos, [v * 16 + iota])
        valid = (v * 16 + iota) < my_count
        g16 = lax.shift_right_logical(
            lax.shift_right_logical(ids16, 7) - blk_lo, GRP_SHIFT)
        new = []
        for g in range(NGRP):
            m = (g16 == g) & valid
            mi = m.astype(jnp.int32)
            rank = plsc.cumsum(mi)
            dst = rank + (ptrs[g] - 1)
            plsc.store_scatter(ids_v, [dst], ids16, mask=m)
            plsc.store_scatter(sel2_pos, [dst], pos16, mask=m)
            new.append(ptrs[g] + jnp.sum(mi))
        return tuple(new)

    lax.fori_loop(0, n_vec, place_grp, tuple(offs))

    BPG = 1 << GRP_SHIFT
    for g in range(NGRP):
        lo_g = goff_s[g]
        hi_g = goff_s[g + 1]

        def cnt_blk(v, cs, g=g, lo_g=lo_g, hi_g=hi_g):
            p16 = v * 16 + iota
            ids16 = plsc.load_gather(ids_v, [p16])
            valid = (p16 >= lo_g) & (p16 < hi_g)
            lb16 = lax.shift_right_logical(ids16, 7) - blk_lo
            return tuple(
                cs[j] + jnp.sum(((lb16 == g * BPG + j) & valid)
                                .astype(jnp.int32))
                for j in range(BPG))

        bc = lax.fori_loop(lo_g >> 4, (hi_g + 15) >> 4, cnt_blk,
                           tuple(jnp.int32(0) for _ in range(BPG)))
        for j in range(BPG):
            boff_s[g * BPG + j] = bc[j]

    def prefix(i, acc2):
        c = boff_s[i]
        boff_s[i] = acc2
        return acc2 + c

    total = lax.fori_loop(0, NGRP * BPG, prefix, jnp.int32(0))
    boff_s[NGRP * BPG] = total

    def init_oidx():
        for v in range(OROWS // 16):
            plsc.store_scatter(oidx, [v * 16 + iota],
                               DUMP_BASE + v * 16 + iota)

    init_oidx()

    def fire(blk, buf):
        safe = jnp.minimum(blk, TAIL_BLOCK - 1)

        @pl.when(blk < TAIL_BLOCK)
        def _():
            off = pl.multiple_of(safe * LANE, LANE)
            pltpu.async_copy(embT_hbm.at[:, pl.ds(off, LANE)],
                             cbuf.at[buf], sem_c)

        @pl.when(blk >= TAIL_BLOCK)
        def _():
            pltpu.async_copy(tail_hbm, cbuf.at[buf], sem_c)

    def wait_block(blk, buf):
        safe = jnp.minimum(blk, TAIL_BLOCK - 1)
        off = pl.multiple_of(safe * LANE, LANE)
        pltpu.make_async_copy(embT_hbm.at[:, pl.ds(off, LANE)],
                              cbuf.at[buf], sem_c).wait()

    def flush(row):
        cp = pltpu.async_copy(obuf, out_hbm.at[oidx], sem_o)
        cp.wait()
        init_oidx()

    def build_list(i, m):
        c = boff_s[i + 1] - boff_s[i]

        @pl.when((c > 0) & (i < nblk))
        def _():
            blist_s[m] = i

        return m + jnp.where((c > 0) & (i < nblk), 1, 0)

    n_live = lax.fori_loop(0, NGRP * BPG, build_list, jnp.int32(0))

    RING = 4

    def prefire(n, _):
        fire(blk_lo + blist_s[n], lax.rem(n, RING))
        return _

    lax.fori_loop(0, jnp.minimum(n_live, RING), prefire, jnp.int32(0))

    for g in range(NGRP):
        lo_g = goff_s[g]
        hi_g = goff_s[g + 1]
        ptr0 = tuple(boff_s[g * BPG + j] for j in range(BPG))

        def place_blk(v, ps, g=g, lo_g=lo_g, hi_g=hi_g):
            p16 = v * 16 + iota
            ids16 = plsc.load_gather(ids_v, [p16])
            pos16 = plsc.load_gather(sel2_pos, [p16])
            valid = (p16 >= lo_g) & (p16 < hi_g)
            lb16 = lax.shift_right_logical(ids16, 7) - blk_lo
            new = []
            for j in range(BPG):
                m = (lb16 == g * BPG + j) & valid
                mi = m.astype(jnp.int32)
                rank = plsc.cumsum(mi)
                dst = rank + (ps[j] - 1)
                plsc.store_scatter(sel_id, [dst], ids16, mask=m)
                plsc.store_scatter(sel_pos, [dst], pos16, mask=m)
                new.append(ps[j] + jnp.sum(mi))
            return tuple(new)

        lax.fori_loop(lo_g >> 4, (hi_g + 15) >> 4, place_blk, ptr0)

    def per_block(n, row):
        lb = blist_s[n]
        blk = blk_lo + lb
        slot = lax.rem(n, RING)
        wait_block(blk, slot)
        k_lo = boff_s[lb]
        k_hi = boff_s[lb + 1]

        def per_hit(k, row):
            @pl.when(row >= OROWS)
            def _():
                flush(row)

            row = jnp.where(row >= OROWS, 0, row)
            kv = jnp.full((16,), k, jnp.int32)
            idv = plsc.load_gather(sel_id, [kv])
            psv = plsc.load_gather(sel_pos, [kv])
            lane = idv & (LANE - 1)
            rowv = jnp.full((16,), row, jnp.int32)
            for s in range(EMBED_SIZE // 16):
                vals = plsc.load_gather(
                    cbuf, [jnp.full((16,), slot, jnp.int32),
                           s * 16 + iota, lane])
                plsc.store_scatter(obuf, [rowv, s * 16 + iota], vals)
            plsc.store_scatter(oidx, [rowv], psv, mask=iota == 0)
            return row + 1

        row = lax.fori_loop(k_lo, k_hi, per_hit, row)

        @pl.when(n + RING < n_live)
        def _():
            fire(blk_lo + blist_s[n + RING], slot)

        return row

    row = lax.fori_loop(0, n_live, per_block, jnp.int32(0))
    flush(row)


_emb_gather = pl.kernel(
    _emb_body,
    out_type=jax.ShapeDtypeStruct((OUT_ROWS, LANE), jnp.float32),
    mesh=_SC_MESH,
    compiler_params=pltpu.CompilerParams(needs_layout_passes=False),
    scratch_types=[
        pltpu.VMEM((BATCH,), jnp.int32),
        pltpu.VMEM((BATCH,), jnp.int32),
        pltpu.VMEM((BATCH,), jnp.int32),
        pltpu.VMEM((BATCH,), jnp.int32),
        pltpu.SMEM((NGRP + 1,), jnp.int32),
        pltpu.SMEM((NGRP * (1 << GRP_SHIFT) + 1,), jnp.int32),
        pltpu.SMEM((NGRP * (1 << GRP_SHIFT),), jnp.int32),
        pltpu.VMEM((4, EMBED_SIZE, LANE), jnp.float32),
        pltpu.VMEM((OROWS, LANE), jnp.float32),
        pltpu.VMEM((OROWS,), jnp.int32),
        pltpu.SemaphoreType.DMA,
        pltpu.SemaphoreType.DMA,
    ],
)


def _proj_body(x_ref, w_ref, b_ref, o_ref):
    o_ref[...] = (jnp.dot(w_ref[...], x_ref[...].T,
                          preferred_element_type=jnp.float32) + b_ref[...])


_ROWS_PER_BLK = 2048


def _tc_proj(x, w, b2d):
    return pl.pallas_call(
        _proj_body,
        grid=(BATCH // _ROWS_PER_BLK,),
        in_specs=[
            pl.BlockSpec((_ROWS_PER_BLK, D_FEAT), lambda i: (i, 0)),
            pl.BlockSpec((EMBED_SIZE, D_FEAT), lambda i: (0, 0)),
            pl.BlockSpec((EMBED_SIZE, 1), lambda i: (0, 0)),
        ],
        out_specs=pl.BlockSpec((EMBED_SIZE, _ROWS_PER_BLK), lambda i: (0, i)),
        out_shape=jax.ShapeDtypeStruct((EMBED_SIZE, BATCH), jnp.float32),
    )(x, w, b2d)


def kernel(node_ids_feat, node_ids_embed, feat_table, proj_W, proj_b,
           embed_table):
    ids_f = node_ids_feat.astype(jnp.int32).reshape(NW, N_CHUNK, IDX_CHUNK)
    ids_e = node_ids_embed.astype(jnp.int32)
    emb_T = embed_table.T
    tail = jnp.zeros((EMBED_SIZE, LANE), jnp.float32)
    tail = tail.at[:, :N_EMB - TAIL_BLOCK * LANE].set(
        emb_T[:, TAIL_BLOCK * LANE:])
    out_pad = _emb_gather(ids_e, emb_T, tail)
    gathered = _gather_feat(ids_f, feat_table)
    feat_T = _tc_proj(gathered, proj_W, proj_b.reshape(EMBED_SIZE, 1))
    return (feat_T.T, out_pad[:BATCH, :EMBED_SIZE])
